# baseline (speedup 1.0000x reference)
import functools

import jax
import jax.numpy as jnp
from jax import lax
from jax.experimental import pallas as pl
from jax.experimental.pallas import tpu as pltpu
from jax.experimental.pallas import tpu_sc as plsc

_COMMIT = 0.25
_N = 16384
_K = 8192
_D = 256
_BLK = 256
_GRID = _N // _BLK


def _mm(a, b):
    return jax.lax.dot(a, b, precision=None)


_ERFC_P = (2.326819970068386E-2, -1.387039388740657E-1, 3.687424674597105E-1,
           -5.824733027278666E-1, 6.210004621745983E-1, -4.944515323274145E-1,
           3.404879937665872E-1, -2.741127028184656E-1, 5.638259427386472E-1)
_ERFC_R = (-1.047766399936249E+1, 1.297719955372516E+1, -7.495518717768503E+0,
           2.921019019210786E+0, -1.015265279202700E+0, 4.218463358204948E-1,
           -2.820767439740514E-1, 5.641895067754075E-1)
_ERF_T = (7.853861353153693E-5, -8.010193625184903E-4, 5.188327685732524E-3,
          -2.685381193529856E-2, 1.128358514861418E-1, -3.761262582423300E-1,
          1.128379165726710E+0)


def _horner(y, cs):
    p = jnp.full_like(y, jnp.float32(cs[0]))
    for c in cs[1:]:
        p = p * y + jnp.float32(c)
    return p


def _erfc(t):
    abs_t = jnp.abs(t)
    z = jnp.exp(-t * t)
    q = 1.0 / abs_t
    y = q * q
    p = jnp.where(abs_t < 2.0, _horner(y, _ERFC_P), _horner(y, _ERFC_R))
    yv = z * q * p
    impl = jnp.where(t < 0.0, 2.0 - yv, yv)
    erf_small = t * _horner(t * t, _ERF_T)
    return jnp.where(abs_t > 1.0, impl, 1.0 - erf_small)


def _gelu(x):
    return 0.5 * x * _erfc(-x * 0.7071067811865476)


def _layernorm(x, g, b):
    m = jnp.mean(x, axis=-1, keepdims=True)
    v = jnp.var(x, axis=-1, keepdims=True)
    return (x - m) / jnp.sqrt(v + 1e-5) * g + b


def _enc_body(x_ref, w0_ref, b0_ref, g0_ref, be0_ref, w1_ref, b1_ref, g1_ref,
              be1_ref, w2_ref, b2_ref, cbt_ref, ze_ref, idx_ref):
    h = _mm(x_ref[...], w0_ref[...]) + b0_ref[...]
    h = _gelu(_layernorm(h, g0_ref[...], be0_ref[...]))
    h = _mm(h, w1_ref[...]) + b1_ref[...]
    h = _gelu(_layernorm(h, g1_ref[...], be1_ref[...]))
    z = _mm(h, w2_ref[...]) + b2_ref[...]
    ze_ref[...] = z

    cbt = cbt_ref[...]
    scores = _mm(z, cbt)
    znorm = jnp.sum(z * z, axis=1, keepdims=True)
    cnorm = jnp.sum(cbt * cbt, axis=0)
    dist = znorm - 2.0 * scores + cnorm[None, :]
    mins = jnp.min(dist, axis=1, keepdims=True)
    cols = lax.broadcasted_iota(jnp.int32, dist.shape, 1)
    idx = jnp.min(jnp.where(dist == mins, cols, jnp.int32(_K)), axis=1)
    idx_ref[...] = idx[:, None]


def _dec_body(zq_ref, ze_ref, x_ref, w0_ref, b0_ref, g0_ref, be0_ref, w1_ref,
              b1_ref, g1_ref, be1_ref, w2_ref, b2_ref, xr_ref, zqst_ref,
              rec_ref, vq_ref):
    zq = zq_ref[...]
    ze = ze_ref[...]
    zqst = ze + (zq - ze)
    zqst_ref[...] = zqst
    h = _mm(zqst, w0_ref[...]) + b0_ref[...]
    h = _gelu(_layernorm(h, g0_ref[...], be0_ref[...]))
    h = _mm(h, w1_ref[...]) + b1_ref[...]
    h = _gelu(_layernorm(h, g1_ref[...], be1_ref[...]))
    xr = _mm(h, w2_ref[...]) + b2_ref[...]
    xr_ref[...] = xr

    ones = jnp.ones((1, 1, 128), jnp.float32)
    rec_ref[...] = jnp.sum((xr - x_ref[...]) ** 2) * ones
    vq_ref[...] = jnp.sum((zq - ze) ** 2) * ones


def _full(shape):
    zeros = (0,) * len(shape)
    return pl.BlockSpec(shape, lambda i: zeros)


def _encode_quantize(x, w0, b0, g0, be0, w1, b1, g1, be1, w2, b2, cbt):
    return pl.pallas_call(
        _enc_body,
        grid=(_GRID,),
        in_specs=[
            pl.BlockSpec((_BLK, 768), lambda i: (i, 0)),
            _full((768, 512)), _full((512,)), _full((512,)), _full((512,)),
            _full((512, 384)), _full((384,)), _full((384,)), _full((384,)),
            _full((384, 256)), _full((256,)),
            _full((_D, _K)),
        ],
        out_specs=[
            pl.BlockSpec((_BLK, _D), lambda i: (i, 0)),
            pl.BlockSpec((_BLK, 1), lambda i: (i, 0)),
        ],
        out_shape=[
            jax.ShapeDtypeStruct((_N, _D), jnp.float32),
            jax.ShapeDtypeStruct((_N, 1), jnp.int32),
        ],
        compiler_params=pltpu.CompilerParams(
            dimension_semantics=("parallel",)),
    )(x, w0, b0, g0, be0, w1, b1, g1, be1, w2, b2, cbt)


def _sc_gather(codebook, idx):
    info = plsc.get_sparse_core_info()
    nw = info.num_cores * info.num_subcores
    b_per_w = _N // nw
    chunk = 128
    nchunk = b_per_w // chunk
    mesh = plsc.VectorSubcoreMesh(core_axis_name="c", subcore_axis_name="s")

    @functools.partial(
        pl.kernel, mesh=mesh,
        out_type=jax.ShapeDtypeStruct((_N, _D), jnp.float32),
        scratch_types=[
            pltpu.VMEM((chunk,), jnp.int32),
            pltpu.VMEM((chunk, _D), jnp.float32),
            pltpu.SemaphoreType.DMA,
        ],
    )
    def gather(table_hbm, idx_hbm, out_hbm, idx_v, rows_v, sem):
        wid = lax.axis_index("s") * info.num_cores + lax.axis_index("c")
        for j in range(nchunk):
            base = wid * b_per_w + j * chunk
            pltpu.sync_copy(idx_hbm.at[pl.ds(base, chunk)], idx_v)
            pltpu.async_copy(table_hbm.at[idx_v], rows_v, sem).wait()
            pltpu.sync_copy(rows_v, out_hbm.at[pl.ds(base, chunk)])

    return gather(codebook, idx)


def _decode(zq, ze, x, w0, b0, g0, be0, w1, b1, g1, be1, w2, b2):
    return pl.pallas_call(
        _dec_body,
        grid=(_GRID,),
        in_specs=[
            pl.BlockSpec((_BLK, _D), lambda i: (i, 0)),
            pl.BlockSpec((_BLK, _D), lambda i: (i, 0)),
            pl.BlockSpec((_BLK, 768), lambda i: (i, 0)),
            _full((256, 384)), _full((384,)), _full((384,)), _full((384,)),
            _full((384, 512)), _full((512,)), _full((512,)), _full((512,)),
            _full((512, 768)), _full((768,)),
        ],
        out_specs=[
            pl.BlockSpec((_BLK, 768), lambda i: (i, 0)),
            pl.BlockSpec((_BLK, _D), lambda i: (i, 0)),
            pl.BlockSpec((1, 1, 128), lambda i: (i, 0, 0)),
            pl.BlockSpec((1, 1, 128), lambda i: (i, 0, 0)),
        ],
        out_shape=[
            jax.ShapeDtypeStruct((_N, 768), jnp.float32),
            jax.ShapeDtypeStruct((_N, _D), jnp.float32),
            jax.ShapeDtypeStruct((_GRID, 1, 128), jnp.float32),
            jax.ShapeDtypeStruct((_GRID, 1, 128), jnp.float32),
        ],
        compiler_params=pltpu.CompilerParams(
            dimension_semantics=("parallel",)),
    )(zq, ze, x, w0, b0, g0, be0, w1, b1, g1, be1, w2, b2)


def kernel(x, enc_W0, enc_b0, enc_g0, enc_be0, enc_W1, enc_b1, enc_g1, enc_be1,
           enc_W2, enc_b2, codebook, dec_W0, dec_b0, dec_g0, dec_be0, dec_W1,
           dec_b1, dec_g1, dec_be1, dec_W2, dec_b2):
    ze, idx2 = _encode_quantize(x, enc_W0, enc_b0, enc_g0, enc_be0, enc_W1,
                                enc_b1, enc_g1, enc_be1, enc_W2, enc_b2,
                                codebook.T)
    idx = idx2.reshape(_N)
    zq = _sc_gather(codebook, idx)
    xr, zqst, rec_sum, vq_sum = _decode(zq, ze, x, dec_W0, dec_b0, dec_g0,
                                        dec_be0, dec_W1, dec_b1, dec_g1,
                                        dec_be1, dec_W2, dec_b2)
    recon_loss = (jnp.sum(rec_sum[:, 0, 0]) / (_N * 768)).reshape(())
    vq_loss = (_COMMIT * (jnp.sum(vq_sum[:, 0, 0]) / (_N * _D))).reshape(())
    total_loss = recon_loss + vq_loss
    return (xr, total_loss, recon_loss, vq_loss, idx, ze, zqst)


# speedup vs baseline: 1.3388x; 1.3388x over previous
import functools

import jax
import jax.numpy as jnp
from jax import lax
from jax.experimental import pallas as pl
from jax.experimental.pallas import tpu as pltpu
from jax.experimental.pallas import tpu_sc as plsc

_COMMIT = 0.25
_N = 16384
_K = 8192
_D = 256
_BLK = 256
_GRID = _N // _BLK


def _mm(a, b):
    return jax.lax.dot(a, b, precision=None)


_ERFC_P = (2.326819970068386E-2, -1.387039388740657E-1, 3.687424674597105E-1,
           -5.824733027278666E-1, 6.210004621745983E-1, -4.944515323274145E-1,
           3.404879937665872E-1, -2.741127028184656E-1, 5.638259427386472E-1)
_ERFC_R = (-1.047766399936249E+1, 1.297719955372516E+1, -7.495518717768503E+0,
           2.921019019210786E+0, -1.015265279202700E+0, 4.218463358204948E-1,
           -2.820767439740514E-1, 5.641895067754075E-1)
_ERF_T = (7.853861353153693E-5, -8.010193625184903E-4, 5.188327685732524E-3,
          -2.685381193529856E-2, 1.128358514861418E-1, -3.761262582423300E-1,
          1.128379165726710E+0)


def _horner(y, cs):
    p = jnp.full_like(y, jnp.float32(cs[0]))
    for c in cs[1:]:
        p = p * y + jnp.float32(c)
    return p


def _erfc(t):
    abs_t = jnp.abs(t)
    z = jnp.exp(-t * t)
    q = 1.0 / abs_t
    y = q * q
    p = jnp.where(abs_t < 2.0, _horner(y, _ERFC_P), _horner(y, _ERFC_R))
    yv = z * q * p
    impl = jnp.where(t < 0.0, 2.0 - yv, yv)
    erf_small = t * _horner(t * t, _ERF_T)
    return jnp.where(abs_t > 1.0, impl, 1.0 - erf_small)


def _gelu(x):
    return 0.5 * x * _erfc(-x * 0.7071067811865476)


def _gelu_erf(x):
    return 0.5 * x * (1.0 + lax.erf(x * 0.7071067811865476))


def _layernorm(x, g, b):
    m = jnp.mean(x, axis=-1, keepdims=True)
    v = jnp.var(x, axis=-1, keepdims=True)
    return (x - m) / jnp.sqrt(v + 1e-5) * g + b


def _enc_body(x_ref, w0_ref, b0_ref, g0_ref, be0_ref, w1_ref, b1_ref, g1_ref,
              be1_ref, w2_ref, b2_ref, cbt_ref, ze_ref, idx_ref):
    h = _mm(x_ref[...], w0_ref[...]) + b0_ref[...]
    h = _gelu_erf(_layernorm(h, g0_ref[...], be0_ref[...]))
    h = _mm(h, w1_ref[...]) + b1_ref[...]
    h = _gelu_erf(_layernorm(h, g1_ref[...], be1_ref[...]))
    z = _mm(h, w2_ref[...]) + b2_ref[...]
    ze_ref[...] = z

    cbt = cbt_ref[...]
    scores = _mm(z, cbt)
    znorm = jnp.sum(z * z, axis=1, keepdims=True)
    cnorm = jnp.sum(cbt * cbt, axis=0)
    dist = znorm - 2.0 * scores + cnorm[None, :]
    mins = jnp.min(dist, axis=1, keepdims=True)
    cols = lax.broadcasted_iota(jnp.int32, dist.shape, 1)
    idx = jnp.min(jnp.where(dist == mins, cols, jnp.int32(_K)), axis=1)
    idx_ref[...] = idx[:, None]


def _dec_body(zq_ref, ze_ref, x_ref, w0_ref, b0_ref, g0_ref, be0_ref, w1_ref,
              b1_ref, g1_ref, be1_ref, w2_ref, b2_ref, xr_ref, zqst_ref,
              rec_ref, vq_ref):
    zq = zq_ref[...]
    ze = ze_ref[...]
    zqst = ze + (zq - ze)
    zqst_ref[...] = zqst
    h = _mm(zqst, w0_ref[...]) + b0_ref[...]
    h = _gelu_erf(_layernorm(h, g0_ref[...], be0_ref[...]))
    h = _mm(h, w1_ref[...]) + b1_ref[...]
    h = _gelu_erf(_layernorm(h, g1_ref[...], be1_ref[...]))
    xr = _mm(h, w2_ref[...]) + b2_ref[...]
    xr_ref[...] = xr

    ones = jnp.ones((1, 1, 128), jnp.float32)
    rec_ref[...] = jnp.sum((xr - x_ref[...]) ** 2) * ones
    vq_ref[...] = jnp.sum((zq - ze) ** 2) * ones


def _full(shape):
    zeros = (0,) * len(shape)
    return pl.BlockSpec(shape, lambda i: zeros)


def _encode_quantize(x, w0, b0, g0, be0, w1, b1, g1, be1, w2, b2, cbt):
    return pl.pallas_call(
        _enc_body,
        grid=(_GRID,),
        in_specs=[
            pl.BlockSpec((_BLK, 768), lambda i: (i, 0)),
            _full((768, 512)), _full((512,)), _full((512,)), _full((512,)),
            _full((512, 384)), _full((384,)), _full((384,)), _full((384,)),
            _full((384, 256)), _full((256,)),
            _full((_D, _K)),
        ],
        out_specs=[
            pl.BlockSpec((_BLK, _D), lambda i: (i, 0)),
            pl.BlockSpec((_BLK, 1), lambda i: (i, 0)),
        ],
        out_shape=[
            jax.ShapeDtypeStruct((_N, _D), jnp.float32),
            jax.ShapeDtypeStruct((_N, 1), jnp.int32),
        ],
        compiler_params=pltpu.CompilerParams(
            dimension_semantics=("parallel",)),
    )(x, w0, b0, g0, be0, w1, b1, g1, be1, w2, b2, cbt)


def _sc_gather(codebook, idx):
    info = plsc.get_sparse_core_info()
    nw = info.num_cores * info.num_subcores
    b_per_w = _N // nw
    chunk = 128
    nchunk = b_per_w // chunk
    mesh = plsc.VectorSubcoreMesh(core_axis_name="c", subcore_axis_name="s")

    @functools.partial(
        pl.kernel, mesh=mesh,
        out_type=jax.ShapeDtypeStruct((_N, _D), jnp.float32),
        scratch_types=[
            pltpu.VMEM((chunk,), jnp.int32),
            pltpu.VMEM((chunk, _D), jnp.float32),
            pltpu.SemaphoreType.DMA,
        ],
    )
    def gather(table_hbm, idx_hbm, out_hbm, idx_v, rows_v, sem):
        wid = lax.axis_index("s") * info.num_cores + lax.axis_index("c")
        for j in range(nchunk):
            base = wid * b_per_w + j * chunk
            pltpu.sync_copy(idx_hbm.at[pl.ds(base, chunk)], idx_v)
            pltpu.async_copy(table_hbm.at[idx_v], rows_v, sem).wait()
            pltpu.sync_copy(rows_v, out_hbm.at[pl.ds(base, chunk)])

    return gather(codebook, idx)


def _decode(zq, ze, x, w0, b0, g0, be0, w1, b1, g1, be1, w2, b2):
    return pl.pallas_call(
        _dec_body,
        grid=(_GRID,),
        in_specs=[
            pl.BlockSpec((_BLK, _D), lambda i: (i, 0)),
            pl.BlockSpec((_BLK, _D), lambda i: (i, 0)),
            pl.BlockSpec((_BLK, 768), lambda i: (i, 0)),
            _full((256, 384)), _full((384,)), _full((384,)), _full((384,)),
            _full((384, 512)), _full((512,)), _full((512,)), _full((512,)),
            _full((512, 768)), _full((768,)),
        ],
        out_specs=[
            pl.BlockSpec((_BLK, 768), lambda i: (i, 0)),
            pl.BlockSpec((_BLK, _D), lambda i: (i, 0)),
            pl.BlockSpec((1, 1, 128), lambda i: (i, 0, 0)),
            pl.BlockSpec((1, 1, 128), lambda i: (i, 0, 0)),
        ],
        out_shape=[
            jax.ShapeDtypeStruct((_N, 768), jnp.float32),
            jax.ShapeDtypeStruct((_N, _D), jnp.float32),
            jax.ShapeDtypeStruct((_GRID, 1, 128), jnp.float32),
            jax.ShapeDtypeStruct((_GRID, 1, 128), jnp.float32),
        ],
        compiler_params=pltpu.CompilerParams(
            dimension_semantics=("parallel",)),
    )(zq, ze, x, w0, b0, g0, be0, w1, b1, g1, be1, w2, b2)


def kernel(x, enc_W0, enc_b0, enc_g0, enc_be0, enc_W1, enc_b1, enc_g1, enc_be1,
           enc_W2, enc_b2, codebook, dec_W0, dec_b0, dec_g0, dec_be0, dec_W1,
           dec_b1, dec_g1, dec_be1, dec_W2, dec_b2):
    ze, idx2 = _encode_quantize(x, enc_W0, enc_b0, enc_g0, enc_be0, enc_W1,
                                enc_b1, enc_g1, enc_be1, enc_W2, enc_b2,
                                codebook.T)
    idx = idx2.reshape(_N)
    zq = _sc_gather(codebook, idx)
    xr, zqst, rec_sum, vq_sum = _decode(zq, ze, x, dec_W0, dec_b0, dec_g0,
                                        dec_be0, dec_W1, dec_b1, dec_g1,
                                        dec_be1, dec_W2, dec_b2)
    recon_loss = (jnp.sum(rec_sum[:, 0, 0]) / (_N * 768)).reshape(())
    vq_loss = (_COMMIT * (jnp.sum(vq_sum[:, 0, 0]) / (_N * _D))).reshape(())
    total_loss = recon_loss + vq_loss
    return (xr, total_loss, recon_loss, vq_loss, idx, ze, zqst)


# speedup vs baseline: 1.4344x; 1.0714x over previous
import functools

import jax
import jax.numpy as jnp
from jax import lax
from jax.experimental import pallas as pl
from jax.experimental.pallas import tpu as pltpu
from jax.experimental.pallas import tpu_sc as plsc

_COMMIT = 0.25
_N = 16384
_K = 8192
_D = 256
_BLK = 256
_GRID = _N // _BLK


def _mm(a, b):
    return jax.lax.dot(a, b, precision=None)


_ERFC_P = (2.326819970068386E-2, -1.387039388740657E-1, 3.687424674597105E-1,
           -5.824733027278666E-1, 6.210004621745983E-1, -4.944515323274145E-1,
           3.404879937665872E-1, -2.741127028184656E-1, 5.638259427386472E-1)
_ERFC_R = (-1.047766399936249E+1, 1.297719955372516E+1, -7.495518717768503E+0,
           2.921019019210786E+0, -1.015265279202700E+0, 4.218463358204948E-1,
           -2.820767439740514E-1, 5.641895067754075E-1)
_ERF_T = (7.853861353153693E-5, -8.010193625184903E-4, 5.188327685732524E-3,
          -2.685381193529856E-2, 1.128358514861418E-1, -3.761262582423300E-1,
          1.128379165726710E+0)


def _horner(y, cs):
    p = jnp.full_like(y, jnp.float32(cs[0]))
    for c in cs[1:]:
        p = p * y + jnp.float32(c)
    return p


def _erfc(t):
    abs_t = jnp.abs(t)
    z = jnp.exp(-t * t)
    q = 1.0 / abs_t
    y = q * q
    p = jnp.where(abs_t < 2.0, _horner(y, _ERFC_P), _horner(y, _ERFC_R))
    yv = z * q * p
    impl = jnp.where(t < 0.0, 2.0 - yv, yv)
    erf_small = t * _horner(t * t, _ERF_T)
    return jnp.where(abs_t > 1.0, impl, 1.0 - erf_small)


def _gelu(x):
    return 0.5 * x * _erfc(-x * 0.7071067811865476)


def _gelu_erf(x):
    return 0.5 * x * (1.0 + lax.erf(x * 0.7071067811865476))


def _layernorm(x, g, b):
    m = jnp.mean(x, axis=-1, keepdims=True)
    v = jnp.var(x, axis=-1, keepdims=True)
    return (x - m) / jnp.sqrt(v + 1e-5) * g + b


def _enc_body(x_ref, w0_ref, b0_ref, g0_ref, be0_ref, w1_ref, b1_ref, g1_ref,
              be1_ref, w2_ref, b2_ref, cbt_ref, cn_ref, ze_ref, idx_ref):
    h = _mm(x_ref[...], w0_ref[...]) + b0_ref[...]
    h = _gelu_erf(_layernorm(h, g0_ref[...], be0_ref[...]))
    h = _mm(h, w1_ref[...]) + b1_ref[...]
    h = _gelu_erf(_layernorm(h, g1_ref[...], be1_ref[...]))
    z = _mm(h, w2_ref[...]) + b2_ref[...]
    ze_ref[...] = z

    scores = _mm(z, cbt_ref[...])
    znorm = jnp.sum(z * z, axis=1, keepdims=True)
    dist = znorm - 2.0 * scores + cn_ref[...]
    mins = jnp.min(dist, axis=1, keepdims=True)
    cols = lax.broadcasted_iota(jnp.int32, dist.shape, 1)
    idx = jnp.min(jnp.where(dist == mins, cols, jnp.int32(_K)), axis=1)
    idx_ref[...] = idx[:, None]


def _dec_body(zq_ref, ze_ref, x_ref, w0_ref, b0_ref, g0_ref, be0_ref, w1_ref,
              b1_ref, g1_ref, be1_ref, w2_ref, b2_ref, xr_ref, zqst_ref,
              rec_ref, vq_ref):
    zq = zq_ref[...]
    ze = ze_ref[...]
    zqst = ze + (zq - ze)
    zqst_ref[...] = zqst
    h = _mm(zqst, w0_ref[...]) + b0_ref[...]
    h = _gelu_erf(_layernorm(h, g0_ref[...], be0_ref[...]))
    h = _mm(h, w1_ref[...]) + b1_ref[...]
    h = _gelu_erf(_layernorm(h, g1_ref[...], be1_ref[...]))
    xr = _mm(h, w2_ref[...]) + b2_ref[...]
    xr_ref[...] = xr

    ones = jnp.ones((1, 1, 128), jnp.float32)
    rec_ref[...] = jnp.sum((xr - x_ref[...]) ** 2) * ones
    vq_ref[...] = jnp.sum((zq - ze) ** 2) * ones


def _cnorm_body(cbt_ref, out_ref):
    cbt = cbt_ref[...]
    out_ref[...] = jnp.sum(cbt * cbt, axis=0, keepdims=True)


def _cnorm(cbt):
    return pl.pallas_call(
        _cnorm_body,
        out_shape=jax.ShapeDtypeStruct((1, _K), jnp.float32),
    )(cbt)


def _full(shape):
    zeros = (0,) * len(shape)
    return pl.BlockSpec(shape, lambda i: zeros)


def _encode_quantize(x, w0, b0, g0, be0, w1, b1, g1, be1, w2, b2, cbt, cn):
    return pl.pallas_call(
        _enc_body,
        grid=(_GRID,),
        in_specs=[
            pl.BlockSpec((_BLK, 768), lambda i: (i, 0)),
            _full((768, 512)), _full((512,)), _full((512,)), _full((512,)),
            _full((512, 384)), _full((384,)), _full((384,)), _full((384,)),
            _full((384, 256)), _full((256,)),
            _full((_D, _K)), _full((1, _K)),
        ],
        out_specs=[
            pl.BlockSpec((_BLK, _D), lambda i: (i, 0)),
            pl.BlockSpec((_BLK, 1), lambda i: (i, 0)),
        ],
        out_shape=[
            jax.ShapeDtypeStruct((_N, _D), jnp.float32),
            jax.ShapeDtypeStruct((_N, 1), jnp.int32),
        ],
        compiler_params=pltpu.CompilerParams(
            dimension_semantics=("parallel",)),
    )(x, w0, b0, g0, be0, w1, b1, g1, be1, w2, b2, cbt, cn)


def _sc_gather(codebook, idx):
    info = plsc.get_sparse_core_info()
    nw = info.num_cores * info.num_subcores
    b_per_w = _N // nw
    chunk = 128
    nchunk = b_per_w // chunk
    mesh = plsc.VectorSubcoreMesh(core_axis_name="c", subcore_axis_name="s")

    @functools.partial(
        pl.kernel, mesh=mesh,
        out_type=jax.ShapeDtypeStruct((_N, _D), jnp.float32),
        scratch_types=[
            pltpu.VMEM((chunk,), jnp.int32),
            pltpu.VMEM((chunk,), jnp.int32),
            pltpu.VMEM((chunk, _D), jnp.float32),
            pltpu.VMEM((chunk, _D), jnp.float32),
            pltpu.SemaphoreType.DMA,
            pltpu.SemaphoreType.DMA,
        ],
    )
    def gather(table_hbm, idx_hbm, out_hbm, idx0, idx1, rows0, rows1,
               sem0, sem1):
        wid = lax.axis_index("s") * info.num_cores + lax.axis_index("c")
        base = wid * b_per_w
        idx_v = (idx0, idx1)
        rows_v = (rows0, rows1)
        sems = (sem0, sem1)
        copies = [None, None]
        for j in range(nchunk):
            b = j % 2
            if copies[b] is not None:
                copies[b].wait()
                pltpu.sync_copy(rows_v[b],
                                out_hbm.at[pl.ds(base + (j - 2) * chunk, chunk)])
            pltpu.sync_copy(idx_hbm.at[pl.ds(base + j * chunk, chunk)],
                            idx_v[b])
            copies[b] = pltpu.async_copy(table_hbm.at[idx_v[b]], rows_v[b],
                                         sems[b])
        for j in range(nchunk - 2, nchunk):
            b = j % 2
            copies[b].wait()
            pltpu.sync_copy(rows_v[b],
                            out_hbm.at[pl.ds(base + j * chunk, chunk)])

    return gather(codebook, idx)


def _decode(zq, ze, x, w0, b0, g0, be0, w1, b1, g1, be1, w2, b2):
    return pl.pallas_call(
        _dec_body,
        grid=(_GRID,),
        in_specs=[
            pl.BlockSpec((_BLK, _D), lambda i: (i, 0)),
            pl.BlockSpec((_BLK, _D), lambda i: (i, 0)),
            pl.BlockSpec((_BLK, 768), lambda i: (i, 0)),
            _full((256, 384)), _full((384,)), _full((384,)), _full((384,)),
            _full((384, 512)), _full((512,)), _full((512,)), _full((512,)),
            _full((512, 768)), _full((768,)),
        ],
        out_specs=[
            pl.BlockSpec((_BLK, 768), lambda i: (i, 0)),
            pl.BlockSpec((_BLK, _D), lambda i: (i, 0)),
            pl.BlockSpec((1, 1, 128), lambda i: (i, 0, 0)),
            pl.BlockSpec((1, 1, 128), lambda i: (i, 0, 0)),
        ],
        out_shape=[
            jax.ShapeDtypeStruct((_N, 768), jnp.float32),
            jax.ShapeDtypeStruct((_N, _D), jnp.float32),
            jax.ShapeDtypeStruct((_GRID, 1, 128), jnp.float32),
            jax.ShapeDtypeStruct((_GRID, 1, 128), jnp.float32),
        ],
        compiler_params=pltpu.CompilerParams(
            dimension_semantics=("parallel",)),
    )(zq, ze, x, w0, b0, g0, be0, w1, b1, g1, be1, w2, b2)


def kernel(x, enc_W0, enc_b0, enc_g0, enc_be0, enc_W1, enc_b1, enc_g1, enc_be1,
           enc_W2, enc_b2, codebook, dec_W0, dec_b0, dec_g0, dec_be0, dec_W1,
           dec_b1, dec_g1, dec_be1, dec_W2, dec_b2):
    cbt = codebook.T
    ze, idx2 = _encode_quantize(x, enc_W0, enc_b0, enc_g0, enc_be0, enc_W1,
                                enc_b1, enc_g1, enc_be1, enc_W2, enc_b2,
                                cbt, _cnorm(cbt))
    idx = idx2.reshape(_N)
    zq = _sc_gather(codebook, idx)
    xr, zqst, rec_sum, vq_sum = _decode(zq, ze, x, dec_W0, dec_b0, dec_g0,
                                        dec_be0, dec_W1, dec_b1, dec_g1,
                                        dec_be1, dec_W2, dec_b2)
    recon_loss = (jnp.sum(rec_sum[:, 0, 0]) / (_N * 768)).reshape(())
    vq_loss = (_COMMIT * (jnp.sum(vq_sum[:, 0, 0]) / (_N * _D))).reshape(())
    total_loss = recon_loss + vq_loss
    return (xr, total_loss, recon_loss, vq_loss, idx, ze, zqst)


# speedup vs baseline: 1.6116x; 1.1235x over previous
import functools

import jax
import jax.numpy as jnp
from jax import lax
from jax.experimental import pallas as pl
from jax.experimental.pallas import tpu as pltpu
from jax.experimental.pallas import tpu_sc as plsc

_COMMIT = 0.25
_N = 16384
_K = 8192
_D = 256
_BLK = 512
_GRID = _N // _BLK


def _mm(a, b):
    return jax.lax.dot(a, b, precision=None)


_ERFC_P = (2.326819970068386E-2, -1.387039388740657E-1, 3.687424674597105E-1,
           -5.824733027278666E-1, 6.210004621745983E-1, -4.944515323274145E-1,
           3.404879937665872E-1, -2.741127028184656E-1, 5.638259427386472E-1)
_ERFC_R = (-1.047766399936249E+1, 1.297719955372516E+1, -7.495518717768503E+0,
           2.921019019210786E+0, -1.015265279202700E+0, 4.218463358204948E-1,
           -2.820767439740514E-1, 5.641895067754075E-1)
_ERF_T = (7.853861353153693E-5, -8.010193625184903E-4, 5.188327685732524E-3,
          -2.685381193529856E-2, 1.128358514861418E-1, -3.761262582423300E-1,
          1.128379165726710E+0)


def _horner(y, cs):
    p = jnp.full_like(y, jnp.float32(cs[0]))
    for c in cs[1:]:
        p = p * y + jnp.float32(c)
    return p


def _erfc(t):
    abs_t = jnp.abs(t)
    z = jnp.exp(-t * t)
    q = 1.0 / abs_t
    y = q * q
    p = jnp.where(abs_t < 2.0, _horner(y, _ERFC_P), _horner(y, _ERFC_R))
    yv = z * q * p
    impl = jnp.where(t < 0.0, 2.0 - yv, yv)
    erf_small = t * _horner(t * t, _ERF_T)
    return jnp.where(abs_t > 1.0, impl, 1.0 - erf_small)


def _gelu(x):
    return 0.5 * x * _erfc(-x * 0.7071067811865476)


def _gelu_erf(x):
    return 0.5 * x * (1.0 + lax.erf(x * 0.7071067811865476))


def _layernorm(x, g, b):
    m = jnp.mean(x, axis=-1, keepdims=True)
    v = jnp.var(x, axis=-1, keepdims=True)
    return (x - m) / jnp.sqrt(v + 1e-5) * g + b


def _enc_body(x_ref, w0_ref, b0_ref, g0_ref, be0_ref, w1_ref, b1_ref, g1_ref,
              be1_ref, w2_ref, b2_ref, cbt_ref, cn_ref, ze_ref, idx_ref):
    h = _mm(x_ref[...], w0_ref[...]) + b0_ref[...]
    h = _gelu_erf(_layernorm(h, g0_ref[...], be0_ref[...]))
    h = _mm(h, w1_ref[...]) + b1_ref[...]
    h = _gelu_erf(_layernorm(h, g1_ref[...], be1_ref[...]))
    z = _mm(h, w2_ref[...]) + b2_ref[...]
    ze_ref[...] = z

    scores = _mm(z, cbt_ref[...])
    znorm = jnp.sum(z * z, axis=1, keepdims=True)
    dist = znorm - 2.0 * scores + cn_ref[...]
    mins = jnp.min(dist, axis=1, keepdims=True)
    cols = lax.broadcasted_iota(jnp.int32, dist.shape, 1)
    idx = jnp.min(jnp.where(dist == mins, cols, jnp.int32(_K)), axis=1)
    idx_ref[...] = idx[:, None]


def _dec_body(zq_ref, ze_ref, x_ref, w0_ref, b0_ref, g0_ref, be0_ref, w1_ref,
              b1_ref, g1_ref, be1_ref, w2_ref, b2_ref, xr_ref, zqst_ref,
              rec_ref, vq_ref):
    zq = zq_ref[...]
    ze = ze_ref[...]
    zqst = ze + (zq - ze)
    zqst_ref[...] = zqst
    h = _mm(zqst, w0_ref[...]) + b0_ref[...]
    h = _gelu_erf(_layernorm(h, g0_ref[...], be0_ref[...]))
    h = _mm(h, w1_ref[...]) + b1_ref[...]
    h = _gelu_erf(_layernorm(h, g1_ref[...], be1_ref[...]))
    xr = _mm(h, w2_ref[...]) + b2_ref[...]
    xr_ref[...] = xr

    ones = jnp.ones((1, 1, 128), jnp.float32)
    rec_ref[...] = jnp.sum((xr - x_ref[...]) ** 2) * ones
    vq_ref[...] = jnp.sum((zq - ze) ** 2) * ones


def _cnorm_body(cbt_ref, out_ref):
    cbt = cbt_ref[...]
    out_ref[...] = jnp.sum(cbt * cbt, axis=0, keepdims=True)


def _cnorm(cbt):
    return pl.pallas_call(
        _cnorm_body,
        out_shape=jax.ShapeDtypeStruct((1, _K), jnp.float32),
    )(cbt)


def _full(shape):
    zeros = (0,) * len(shape)
    return pl.BlockSpec(shape, lambda i: zeros)


def _encode_quantize(x, w0, b0, g0, be0, w1, b1, g1, be1, w2, b2, cbt, cn):
    return pl.pallas_call(
        _enc_body,
        grid=(_GRID,),
        in_specs=[
            pl.BlockSpec((_BLK, 768), lambda i: (i, 0)),
            _full((768, 512)), _full((512,)), _full((512,)), _full((512,)),
            _full((512, 384)), _full((384,)), _full((384,)), _full((384,)),
            _full((384, 256)), _full((256,)),
            _full((_D, _K)), _full((1, _K)),
        ],
        out_specs=[
            pl.BlockSpec((_BLK, _D), lambda i: (i, 0)),
            pl.BlockSpec((_BLK, 1), lambda i: (i, 0)),
        ],
        out_shape=[
            jax.ShapeDtypeStruct((_N, _D), jnp.float32),
            jax.ShapeDtypeStruct((_N, 1), jnp.int32),
        ],
        compiler_params=pltpu.CompilerParams(
            dimension_semantics=("parallel",)),
    )(x, w0, b0, g0, be0, w1, b1, g1, be1, w2, b2, cbt, cn)


def _sc_gather(codebook, idx):
    info = plsc.get_sparse_core_info()
    nw = info.num_cores * info.num_subcores
    b_per_w = _N // nw
    chunk = 128
    nchunk = b_per_w // chunk
    mesh = plsc.VectorSubcoreMesh(core_axis_name="c", subcore_axis_name="s")

    @functools.partial(
        pl.kernel, mesh=mesh,
        out_type=jax.ShapeDtypeStruct((_N, _D), jnp.float32),
        scratch_types=[
            pltpu.VMEM((chunk,), jnp.int32),
            pltpu.VMEM((chunk,), jnp.int32),
            pltpu.VMEM((chunk, _D), jnp.float32),
            pltpu.VMEM((chunk, _D), jnp.float32),
            pltpu.SemaphoreType.DMA,
            pltpu.SemaphoreType.DMA,
        ],
    )
    def gather(table_hbm, idx_hbm, out_hbm, idx0, idx1, rows0, rows1,
               sem0, sem1):
        wid = lax.axis_index("s") * info.num_cores + lax.axis_index("c")
        base = wid * b_per_w
        idx_v = (idx0, idx1)
        rows_v = (rows0, rows1)
        sems = (sem0, sem1)
        copies = [None, None]
        for j in range(nchunk):
            b = j % 2
            if copies[b] is not None:
                copies[b].wait()
                pltpu.sync_copy(rows_v[b],
                                out_hbm.at[pl.ds(base + (j - 2) * chunk, chunk)])
            pltpu.sync_copy(idx_hbm.at[pl.ds(base + j * chunk, chunk)],
                            idx_v[b])
            copies[b] = pltpu.async_copy(table_hbm.at[idx_v[b]], rows_v[b],
                                         sems[b])
        for j in range(nchunk - 2, nchunk):
            b = j % 2
            copies[b].wait()
            pltpu.sync_copy(rows_v[b],
                            out_hbm.at[pl.ds(base + j * chunk, chunk)])

    return gather(codebook, idx)


def _decode(zq, ze, x, w0, b0, g0, be0, w1, b1, g1, be1, w2, b2):
    return pl.pallas_call(
        _dec_body,
        grid=(_GRID,),
        in_specs=[
            pl.BlockSpec((_BLK, _D), lambda i: (i, 0)),
            pl.BlockSpec((_BLK, _D), lambda i: (i, 0)),
            pl.BlockSpec((_BLK, 768), lambda i: (i, 0)),
            _full((256, 384)), _full((384,)), _full((384,)), _full((384,)),
            _full((384, 512)), _full((512,)), _full((512,)), _full((512,)),
            _full((512, 768)), _full((768,)),
        ],
        out_specs=[
            pl.BlockSpec((_BLK, 768), lambda i: (i, 0)),
            pl.BlockSpec((_BLK, _D), lambda i: (i, 0)),
            pl.BlockSpec((1, 1, 128), lambda i: (i, 0, 0)),
            pl.BlockSpec((1, 1, 128), lambda i: (i, 0, 0)),
        ],
        out_shape=[
            jax.ShapeDtypeStruct((_N, 768), jnp.float32),
            jax.ShapeDtypeStruct((_N, _D), jnp.float32),
            jax.ShapeDtypeStruct((_GRID, 1, 128), jnp.float32),
            jax.ShapeDtypeStruct((_GRID, 1, 128), jnp.float32),
        ],
        compiler_params=pltpu.CompilerParams(
            dimension_semantics=("parallel",)),
    )(zq, ze, x, w0, b0, g0, be0, w1, b1, g1, be1, w2, b2)


def kernel(x, enc_W0, enc_b0, enc_g0, enc_be0, enc_W1, enc_b1, enc_g1, enc_be1,
           enc_W2, enc_b2, codebook, dec_W0, dec_b0, dec_g0, dec_be0, dec_W1,
           dec_b1, dec_g1, dec_be1, dec_W2, dec_b2):
    cbt = codebook.T
    ze, idx2 = _encode_quantize(x, enc_W0, enc_b0, enc_g0, enc_be0, enc_W1,
                                enc_b1, enc_g1, enc_be1, enc_W2, enc_b2,
                                cbt, _cnorm(cbt))
    idx = idx2.reshape(_N)
    zq = _sc_gather(codebook, idx)
    xr, zqst, rec_sum, vq_sum = _decode(zq, ze, x, dec_W0, dec_b0, dec_g0,
                                        dec_be0, dec_W1, dec_b1, dec_g1,
                                        dec_be1, dec_W2, dec_b2)
    recon_loss = (jnp.sum(rec_sum[:, 0, 0]) / (_N * 768)).reshape(())
    vq_loss = (_COMMIT * (jnp.sum(vq_sum[:, 0, 0]) / (_N * _D))).reshape(())
    total_loss = recon_loss + vq_loss
    return (xr, total_loss, recon_loss, vq_loss, idx, ze, zqst)
